# Initial kernel scaffold; baseline (speedup 1.0000x reference)
import functools

import jax
import jax.numpy as jnp
from jax import lax
from jax.experimental import pallas as pl
from jax.experimental.pallas import tpu as pltpu
from jax.experimental.pallas import tpu_sc as plsc

N_NODES = 10000
N_EDGES = 320000
D_IN = 128
D_HID = 256
D_OUT = 128

NC = 2
NS = 16
LANES = 16

CHUNK = 128
N_PAD = 10240
E_PAD = 323584

ROWS_PER_SUB = N_PAD // NS
ZROWS = 16


def _spmm_kernel_body(edge_split_cores, dcols, x_hbm, src_hbm, dst_hbm, w_hbm,
                      p_hbm, idx_s, idx_d, w_v, rows, zbuf, accum, sem):
    c = lax.axis_index("c")
    s = lax.axis_index("s")
    ngroups = dcols // LANES

    @pl.loop(0, ZROWS)
    def _(i):
        for g in range(ngroups):
            zbuf[i, pl.ds(g * LANES, LANES)] = jnp.zeros((LANES,), jnp.float32)

    @pl.loop(0, ROWS_PER_SUB // ZROWS)
    def _(j):
        pltpu.sync_copy(zbuf, accum.at[pl.ds(s * ROWS_PER_SUB + j * ZROWS, ZROWS), :])

    plsc.subcore_barrier()

    if edge_split_cores:
        wid = s * NC + c
        per_w = E_PAD // (NC * NS)
        row_off = None
    else:
        wid = s
        per_w = E_PAD // NS
        row_off = c * N_NODES

    nchunks = per_w // CHUNK

    @pl.loop(0, nchunks)
    def _(t):
        off = wid * per_w + t * CHUNK
        pltpu.sync_copy(src_hbm.at[pl.ds(off, CHUNK)], idx_s)
        pltpu.sync_copy(dst_hbm.at[pl.ds(off, CHUNK)], idx_d)
        pltpu.sync_copy(w_hbm.at[pl.ds(off, CHUNK)], w_v)
        if row_off is not None:
            for g in range(CHUNK // LANES):
                sl = pl.ds(g * LANES, LANES)
                idx_s[sl] = idx_s[sl] + row_off
        pltpu.async_copy(x_hbm.at[idx_s], rows, sem).wait()

        @pl.loop(0, CHUNK)
        def _(i):
            wt = w_v[i]
            for g in range(ngroups):
                sl = (i, pl.ds(g * LANES, LANES))
                rows[sl] = rows[sl] * wt

        pltpu.sync_copy(rows, accum.at[idx_d], add=True)

    plsc.subcore_barrier()

    pltpu.sync_copy(accum.at[pl.ds(s * ROWS_PER_SUB, ROWS_PER_SUB), :],
                    p_hbm.at[c].at[pl.ds(s * ROWS_PER_SUB, ROWS_PER_SUB), :])


def _make_spmm(edge_split_cores, dcols):
    mesh = plsc.VectorSubcoreMesh(core_axis_name="c", subcore_axis_name="s")
    kern = functools.partial(_spmm_kernel_body, edge_split_cores, dcols)
    return pl.kernel(
        kern,
        out_type=jax.ShapeDtypeStruct((NC, N_PAD, dcols), jnp.float32),
        mesh=mesh,
        scratch_types=[
            pltpu.VMEM((CHUNK,), jnp.int32),
            pltpu.VMEM((CHUNK,), jnp.int32),
            pltpu.VMEM((CHUNK,), jnp.float32),
            pltpu.VMEM((CHUNK, dcols), jnp.float32),
            pltpu.VMEM((ZROWS, dcols), jnp.float32),
            pltpu.VMEM_SHARED((N_PAD, dcols), jnp.float32),
            pltpu.SemaphoreType.DMA,
        ],
    )


_spmm_l1 = _make_spmm(edge_split_cores=True, dcols=128)
_spmm_l2 = _make_spmm(edge_split_cores=False, dcols=128)

_ROWS_BLK = 400


def _mm1(P, W1, b1):
    def body(p_ref, w_ref, b_ref, o_ref):
        z = p_ref[0] + p_ref[1]
        h = jnp.dot(z, w_ref[...], preferred_element_type=jnp.float32)
        h = jnp.maximum(h + b_ref[...], 0.0)
        o_ref[0] = h[:, :128]
        o_ref[1] = h[:, 128:]

    return pl.pallas_call(
        body,
        grid=(N_NODES // _ROWS_BLK,),
        in_specs=[
            pl.BlockSpec((NC, _ROWS_BLK, 128), lambda i: (0, i, 0)),
            pl.BlockSpec((D_IN, D_HID), lambda i: (0, 0)),
            pl.BlockSpec((1, D_HID), lambda i: (0, 0)),
        ],
        out_specs=pl.BlockSpec((NC, _ROWS_BLK, 128), lambda i: (0, i, 0)),
        out_shape=jax.ShapeDtypeStruct((NC, N_NODES, 128), jnp.float32),
    )(P, W1, b1)


def _mm2(Z2, W2r, b2, Wfc, bfc):
    def body(z_ref, w2_ref, b2_ref, wfc_ref, bfc_ref, o_ref):
        h = jnp.dot(z_ref[0], w2_ref[0], preferred_element_type=jnp.float32)
        h = h + jnp.dot(z_ref[1], w2_ref[1], preferred_element_type=jnp.float32)
        h = jnp.maximum(h + b2_ref[...], 0.0)
        o = jnp.dot(h, wfc_ref[...], preferred_element_type=jnp.float32)
        o_ref[...] = o + bfc_ref[...]

    return pl.pallas_call(
        body,
        grid=(N_NODES // _ROWS_BLK,),
        in_specs=[
            pl.BlockSpec((NC, _ROWS_BLK, 128), lambda i: (0, i, 0)),
            pl.BlockSpec((NC, 128, D_HID), lambda i: (0, 0, 0)),
            pl.BlockSpec((1, D_HID), lambda i: (0, 0)),
            pl.BlockSpec((D_HID, D_OUT), lambda i: (0, 0)),
            pl.BlockSpec((1, D_OUT), lambda i: (0, 0)),
        ],
        out_specs=pl.BlockSpec((_ROWS_BLK, D_OUT), lambda i: (i, 0)),
        out_shape=jax.ShapeDtypeStruct((N_NODES, D_OUT), jnp.float32),
    )(Z2, W2r, b2, Wfc, bfc)


def kernel(x, edge_index, edge_weight, W1, b1, W2, b2, Wfc, bfc):
    src = edge_index[0]
    dst = edge_index[1]
    pad = E_PAD - N_EDGES
    src_p = jnp.concatenate([src, jnp.zeros((pad,), src.dtype)])
    dst_p = jnp.concatenate([dst, jnp.zeros((pad,), dst.dtype)])
    w_p = jnp.concatenate([edge_weight, jnp.zeros((pad,), edge_weight.dtype)])

    P = _spmm_l1(x, src_p, dst_p, w_p)
    h1 = _mm1(P, W1, b1.reshape(1, D_HID))
    Z2 = _spmm_l2(h1.reshape(NC * N_NODES, 128), src_p, dst_p, w_p)
    out = _mm2(Z2, W2.reshape(NC, 128, D_HID), b2.reshape(1, D_HID),
               Wfc, bfc.reshape(1, D_OUT))
    return out


# speedup vs baseline: 3.1877x; 3.1877x over previous
import functools

import jax
import jax.numpy as jnp
from jax import lax
from jax.experimental import pallas as pl
from jax.experimental.pallas import tpu as pltpu
from jax.experimental.pallas import tpu_sc as plsc

N_NODES = 10000
N_EDGES = 320000
D_IN = 128
D_HID = 256
D_OUT = 128

NC = 2
NS = 16
LANES = 16

CHUNK = 128
N_PAD = 10240
E_PAD = 323584

ROWS_PER_SUB = N_PAD // NS
ZROWS = 16


def _spmm_kernel_body(edge_split_cores, dcols, x_hbm, src_hbm, dst_hbm, w_hbm,
                      p_hbm, idx_s, idx_d, w_v, rows, zbuf, accum, sem):
    c = lax.axis_index("c")
    s = lax.axis_index("s")
    ngroups = dcols // LANES

    @pl.loop(0, ZROWS)
    def _(i):
        for g in range(ngroups):
            zbuf[i, pl.ds(g * LANES, LANES)] = jnp.zeros((LANES,), jnp.float32)

    @pl.loop(0, ROWS_PER_SUB // ZROWS)
    def _(j):
        pltpu.sync_copy(zbuf, accum.at[pl.ds(s * ROWS_PER_SUB + j * ZROWS, ZROWS), :])

    plsc.subcore_barrier()

    if edge_split_cores:
        wid = s * NC + c
        per_w = E_PAD // (NC * NS)
        row_off = None
    else:
        wid = s
        per_w = E_PAD // NS
        row_off = c * N_NODES

    nchunks = per_w // CHUNK

    @pl.loop(0, nchunks)
    def _(t):
        off = wid * per_w + t * CHUNK
        pltpu.sync_copy(src_hbm.at[pl.ds(off, CHUNK)], idx_s)
        pltpu.sync_copy(dst_hbm.at[pl.ds(off, CHUNK)], idx_d)
        pltpu.sync_copy(w_hbm.at[pl.ds(off, CHUNK)], w_v)
        if row_off is not None:
            for g in range(CHUNK // LANES):
                sl = pl.ds(g * LANES, LANES)
                idx_s[sl] = idx_s[sl] + row_off
        pltpu.async_copy(x_hbm.at[idx_s], rows, sem).wait()

        @pl.loop(0, CHUNK // LANES)
        def _(q):
            wv = w_v[pl.ds(q * LANES, LANES)]
            for j in range(LANES):
                wt = wv[j]
                for g in range(ngroups):
                    sl = (q * LANES + j, pl.ds(g * LANES, LANES))
                    rows[sl] = rows[sl] * wt

        pltpu.sync_copy(rows, accum.at[idx_d], add=True)

    plsc.subcore_barrier()

    pltpu.sync_copy(accum.at[pl.ds(s * ROWS_PER_SUB, ROWS_PER_SUB), :],
                    p_hbm.at[c].at[pl.ds(s * ROWS_PER_SUB, ROWS_PER_SUB), :])


def _make_spmm(edge_split_cores, dcols):
    mesh = plsc.VectorSubcoreMesh(core_axis_name="c", subcore_axis_name="s")
    kern = functools.partial(_spmm_kernel_body, edge_split_cores, dcols)
    return pl.kernel(
        kern,
        out_type=jax.ShapeDtypeStruct((NC, N_PAD, dcols), jnp.float32),
        mesh=mesh,
        scratch_types=[
            pltpu.VMEM((CHUNK,), jnp.int32),
            pltpu.VMEM((CHUNK,), jnp.int32),
            pltpu.VMEM((CHUNK,), jnp.float32),
            pltpu.VMEM((CHUNK, dcols), jnp.float32),
            pltpu.VMEM((ZROWS, dcols), jnp.float32),
            pltpu.VMEM_SHARED((N_PAD, dcols), jnp.float32),
            pltpu.SemaphoreType.DMA,
        ],
    )


_spmm_l1 = _make_spmm(edge_split_cores=True, dcols=128)
_spmm_l2 = _make_spmm(edge_split_cores=False, dcols=128)

_ROWS_BLK = 400


def _mm1(P, W1, b1):
    def body(p_ref, w_ref, b_ref, o_ref):
        z = p_ref[0] + p_ref[1]
        h = jnp.dot(z, w_ref[...], preferred_element_type=jnp.float32)
        h = jnp.maximum(h + b_ref[...], 0.0)
        o_ref[0] = h[:, :128]
        o_ref[1] = h[:, 128:]

    return pl.pallas_call(
        body,
        grid=(N_NODES // _ROWS_BLK,),
        in_specs=[
            pl.BlockSpec((NC, _ROWS_BLK, 128), lambda i: (0, i, 0)),
            pl.BlockSpec((D_IN, D_HID), lambda i: (0, 0)),
            pl.BlockSpec((1, D_HID), lambda i: (0, 0)),
        ],
        out_specs=pl.BlockSpec((NC, _ROWS_BLK, 128), lambda i: (0, i, 0)),
        out_shape=jax.ShapeDtypeStruct((NC, N_NODES, 128), jnp.float32),
    )(P, W1, b1)


def _mm2(Z2, W2r, b2, Wfc, bfc):
    def body(z_ref, w2_ref, b2_ref, wfc_ref, bfc_ref, o_ref):
        h = jnp.dot(z_ref[0], w2_ref[0], preferred_element_type=jnp.float32)
        h = h + jnp.dot(z_ref[1], w2_ref[1], preferred_element_type=jnp.float32)
        h = jnp.maximum(h + b2_ref[...], 0.0)
        o = jnp.dot(h, wfc_ref[...], preferred_element_type=jnp.float32)
        o_ref[...] = o + bfc_ref[...]

    return pl.pallas_call(
        body,
        grid=(N_NODES // _ROWS_BLK,),
        in_specs=[
            pl.BlockSpec((NC, _ROWS_BLK, 128), lambda i: (0, i, 0)),
            pl.BlockSpec((NC, 128, D_HID), lambda i: (0, 0, 0)),
            pl.BlockSpec((1, D_HID), lambda i: (0, 0)),
            pl.BlockSpec((D_HID, D_OUT), lambda i: (0, 0)),
            pl.BlockSpec((1, D_OUT), lambda i: (0, 0)),
        ],
        out_specs=pl.BlockSpec((_ROWS_BLK, D_OUT), lambda i: (i, 0)),
        out_shape=jax.ShapeDtypeStruct((N_NODES, D_OUT), jnp.float32),
    )(Z2, W2r, b2, Wfc, bfc)


def kernel(x, edge_index, edge_weight, W1, b1, W2, b2, Wfc, bfc):
    src = edge_index[0]
    dst = edge_index[1]
    pad = E_PAD - N_EDGES
    src_p = jnp.concatenate([src, jnp.zeros((pad,), src.dtype)])
    dst_p = jnp.concatenate([dst, jnp.zeros((pad,), dst.dtype)])
    w_p = jnp.concatenate([edge_weight, jnp.zeros((pad,), edge_weight.dtype)])

    P = _spmm_l1(x, src_p, dst_p, w_p)
    h1 = _mm1(P, W1, b1.reshape(1, D_HID))
    Z2 = _spmm_l2(h1.reshape(NC * N_NODES, 128), src_p, dst_p, w_p)
    out = _mm2(Z2, W2.reshape(NC, 128, D_HID), b2.reshape(1, D_HID),
               Wfc, bfc.reshape(1, D_OUT))
    return out
